# baseline (speedup 1.0000x reference)
import functools

import jax
import jax.numpy as jnp
from jax.experimental import pallas as pl
from jax.experimental.pallas import tpu as pltpu

N_FRAMES = 128
G = 16
GRID = N_FRAMES // G
V = 431
C = 1280
F1, F2 = 16, 8


def _ln_rows(x3, w_row, b_row, eps=1e-12):
    u = jnp.mean(x3, axis=1, keepdims=True)
    d = x3 - u
    s = jnp.mean(d * d, axis=1, keepdims=True)
    return w_row * (d * jax.lax.rsqrt(s + eps)) + b_row


def _fused_body(ht_ref, w1t_ref, b1_ref, adjt_ref, a1_ref, ag_ref, a2_ref,
                w3t_ref, b3_ref, lnp_w_ref, lnp_b_ref, l1b_ref, ln1w_ref,
                ln1b_ref, gcb_ref, ln2w_ref, ln2b_ref, l2b_ref, out_ref):
    f32 = jnp.float32
    x = jnp.dot(ht_ref[...], w1t_ref[...], preferred_element_type=f32)
    x = x + b1_ref[...]
    x3 = x.reshape(G, F1, V)
    t = jax.nn.relu(_ln_rows(x3, lnp_w_ref[...].reshape(1, F1, 1),
                             lnp_b_ref[...].reshape(1, F1, 1)))
    y = jnp.dot(a1_ref[...], t.reshape(G * F1, V), preferred_element_type=f32)
    y3 = y.reshape(G, F2, V) + l1b_ref[...].reshape(1, F2, 1)
    y3 = jax.nn.relu(_ln_rows(y3, ln1w_ref[...].reshape(1, F2, 1),
                              ln1b_ref[...].reshape(1, F2, 1)))
    y = y3.reshape(G * F2, V)
    gcb = gcb_ref[...].reshape(1, F2, 1)
    for _ in range(2):
        u = jnp.dot(ag_ref[...], y, preferred_element_type=f32)
        y = jnp.dot(u, adjt_ref[...], preferred_element_type=f32)
        y = (y.reshape(G, F2, V) + gcb).reshape(G * F2, V)
    t = jax.nn.relu(_ln_rows(y.reshape(G, F2, V),
                             ln2w_ref[...].reshape(1, F2, 1),
                             ln2b_ref[...].reshape(1, F2, 1)))
    z = jnp.dot(a2_ref[...], t.reshape(G * F2, V), preferred_element_type=f32)
    z = (z.reshape(G, F1, V) + l2b_ref[...].reshape(1, F1, 1)).reshape(G * F1, V)
    z = z + x
    out = jnp.dot(z, w3t_ref[...], preferred_element_type=f32)
    out_ref[...] = out + b3_ref[...]


@functools.partial(jax.jit, static_argnames=())
def kernel(hidden_states, W1, b1, ln_pre_w, ln_pre_b, lin1_w, lin1_b,
           ln1_w, ln1_b, gcn_w, gcn_b, adjmat, ln2_w, ln2_b,
           lin2_w, lin2_b, W3, b3):
    B, _, T = hidden_states.shape[0], hidden_states.shape[1], hidden_states.shape[2]
    ht = hidden_states.reshape(N_FRAMES, C, F1).transpose(0, 2, 1).reshape(
        N_FRAMES * F1, C)
    eye = jnp.eye(G, dtype=jnp.float32)
    a1 = jnp.kron(eye, lin1_w)
    ag = jnp.kron(eye, gcn_w.T)
    a2 = jnp.kron(eye, lin2_w)

    row = lambda v: v.reshape(1, -1)
    full = lambda s: pl.BlockSpec(s, lambda i: (0, 0))
    out2d = pl.pallas_call(
        _fused_body,
        grid=(GRID,),
        in_specs=[
            pl.BlockSpec((G * F1, C), lambda i: (i, 0)),
            full((C, V)),
            full((1, V)),
            full((V, V)),
            full((G * F2, G * F1)),
            full((G * F2, G * F2)),
            full((G * F1, G * F2)),
            full((V, C)),
            full((1, C)),
            full((1, F1)), full((1, F1)),
            full((1, F2)),
            full((1, F2)), full((1, F2)),
            full((1, F2)),
            full((1, F2)), full((1, F2)),
            full((1, F1)),
        ],
        out_specs=pl.BlockSpec((G * F1, C), lambda i: (i, 0)),
        out_shape=jax.ShapeDtypeStruct((N_FRAMES * F1, C), jnp.float32),
    )(ht, W1.T, row(b1), adjmat.T, a1, ag, a2, W3.T, row(b3),
      row(ln_pre_w), row(ln_pre_b), row(lin1_b), row(ln1_w), row(ln1_b),
      row(gcn_b), row(ln2_w), row(ln2_b), row(lin2_b))

    return out2d.reshape(N_FRAMES, F1, C).transpose(0, 2, 1).reshape(
        B, C, T, 4, 4)
